# baseline (speedup 1.0000x reference)
import functools
import math

import jax
import jax.numpy as jnp
from jax import lax
from jax.experimental import pallas as pl
from jax.experimental.pallas import tpu as pltpu
from jax.experimental.pallas import tpu_sc as plsc

KERNEL_SIZE_FACTOR = 0.1
SIGMA = 3.0


def _kernel_consts(H):
    ks = int(H * KERNEL_SIZE_FACTOR)
    if ks % 2 == 0:
        ks += 1
    half = ks // 2
    c = (ks - 1) / 2.0
    z = sum(math.exp(-((i - c) ** 2) / (2.0 * SIGMA**2)) for i in range(ks))
    return ks, half, 1.0 / z


_NC = 2
_NS = 16
_L = 16


def _sc_scatter_counts(points, B, P, H, W):
    BPC = B // _NC
    ZROWS = 16

    mesh = plsc.VectorSubcoreMesh(
        core_axis_name="c", subcore_axis_name="s"
    )

    @functools.partial(
        pl.kernel,
        out_type=jax.ShapeDtypeStruct((B, H, W), jnp.float32),
        mesh=mesh,
        scratch_types=[
            pltpu.VMEM((P, 2), jnp.float32),
            pltpu.VMEM((ZROWS, W), jnp.float32),
            pltpu.VMEM((H, W), jnp.float32),
            pltpu.VMEM_SHARED((ZROWS, W), jnp.float32),
        ],
        compiler_params=pltpu.CompilerParams(
            use_tc_tiling_on_sc=False, needs_layout_passes=False
        ),
    )
    def sc_scatter(pts_hbm, out_hbm, pv, zv, cmap, spz):
        c = lax.axis_index("c")
        s = lax.axis_index("s")

        @pl.when(s < BPC)
        def _():
            pltpu.sync_copy(pts_hbm.at[c * BPC + s], pv)

        @pl.when(s == 0)
        def _():
            def zbody(i, carry):
                zv[i // (W // _L), pl.ds((i % (W // _L)) * _L, _L)] = (
                    jnp.zeros((_L,), jnp.float32)
                )
                return carry

            lax.fori_loop(0, ZROWS * (W // _L), zbody, 0)
            pltpu.sync_copy(zv, spz)

        plsc.subcore_barrier()

        @pl.when(s < BPC)
        def _():
            b = c * BPC + s

            for k in range(H // ZROWS):
                pltpu.sync_copy(spz, cmap.at[pl.ds(k * ZROWS, ZROWS)])

            ones = jnp.ones((_L,), jnp.float32)
            lane = lax.iota(jnp.int32, _L)
            zero = jnp.zeros((_L,), jnp.int32)

            def sbody(i, carry):
                rows = i * _L + lane
                x16 = plsc.load_gather(pv, [rows, zero])
                y16 = plsc.load_gather(pv, [rows, zero + 1])
                xi = (x16 * W).astype(jnp.int32)
                yi = (y16 * H).astype(jnp.int32)
                plsc.addupdate_scatter(cmap, [yi, xi], ones)
                return carry

            lax.fori_loop(0, P // _L, sbody, 0)

            pltpu.sync_copy(cmap, out_hbm.at[b])

    return sc_scatter(points)


def _tc_blur_body(m_ref, o_ref, t_ref, *, H, half, inv_z):
    @pl.when(pl.program_id(0) == 0)
    def _():
        r = lax.broadcasted_iota(jnp.int32, (H, H), 0).astype(jnp.float32)
        cc = lax.broadcasted_iota(jnp.int32, (H, H), 1).astype(jnp.float32)
        d = r - cc
        inv_two_sigma2 = -1.0 / (2.0 * SIGMA * SIGMA)
        t_ref[...] = jnp.where(
            jnp.abs(d) <= half,
            jnp.exp(d * d * inv_two_sigma2) * inv_z,
            0.0,
        )

    T = t_ref[...]
    A = lax.dot_general(
        T, m_ref[0], (((1,), (0,)), ((), ())),
        preferred_element_type=jnp.float32,
    )
    o_ref[0] = lax.dot_general(
        A, T, (((1,), (0,)), ((), ())),
        preferred_element_type=jnp.float32,
    )


def _tc_blur(counts, B, H, W, half, inv_z):
    body = functools.partial(_tc_blur_body, H=H, half=half, inv_z=inv_z)
    return pl.pallas_call(
        body,
        grid=(B,),
        in_specs=[pl.BlockSpec((1, H, W), lambda b: (b, 0, 0))],
        out_specs=pl.BlockSpec((1, H, W), lambda b: (b, 0, 0)),
        out_shape=jax.ShapeDtypeStruct((B, H, W), jnp.float32),
        scratch_shapes=[pltpu.VMEM((H, H), jnp.float32)],
    )(counts)


def kernel(feature_map, points):
    B, C, H, W = feature_map.shape
    P = points.shape[1]
    ks, half, inv_z = _kernel_consts(min(H, W))

    counts = _sc_scatter_counts(points, B, P, H, W)
    return _tc_blur(counts, B, H, W, half, inv_z)


# speedup vs baseline: 1.2854x; 1.2854x over previous
import functools
import math

import jax
import jax.numpy as jnp
from jax import lax
from jax.experimental import pallas as pl
from jax.experimental.pallas import tpu as pltpu
from jax.experimental.pallas import tpu_sc as plsc

KERNEL_SIZE_FACTOR = 0.1
SIGMA = 3.0


def _kernel_consts(H):
    ks = int(H * KERNEL_SIZE_FACTOR)
    if ks % 2 == 0:
        ks += 1
    half = ks // 2
    c = (ks - 1) / 2.0
    z = sum(math.exp(-((i - c) ** 2) / (2.0 * SIGMA**2)) for i in range(ks))
    return ks, half, 1.0 / z


_NC = 2
_NS = 16
_L = 16


def _sc_scatter_counts(pts_t, B, P, H, W):
    MAP = H * W
    BPC = B // _NC
    TPB = _NS // BPC
    PER_TILE = P // TPB
    CORE_MAP = BPC * MAP
    SLICE = CORE_MAP // _NS
    ZCH = SLICE // 4
    NIDX = 128

    mesh = plsc.VectorSubcoreMesh(
        core_axis_name="c", subcore_axis_name="s"
    )

    @functools.partial(
        pl.kernel,
        out_type=jax.ShapeDtypeStruct((B * MAP,), jnp.float32),
        mesh=mesh,
        scratch_types=[
            pltpu.VMEM((PER_TILE,), jnp.float32),
            pltpu.VMEM((PER_TILE,), jnp.float32),
            pltpu.VMEM((ZCH,), jnp.float32),
            pltpu.VMEM((NIDX,), jnp.float32),
            pltpu.VMEM((PER_TILE // NIDX, NIDX), jnp.int32),
            pltpu.VMEM_SHARED((CORE_MAP,), jnp.float32),
        ],
    )
    def sc_scatter(pts_hbm, out_hbm, xv, yv, zv, ones_v, ivm, smap):
        c = lax.axis_index("c")
        s = lax.axis_index("s")
        b_local = s // TPB
        b = c * BPC + b_local
        po = (s % TPB) * PER_TILE

        pltpu.sync_copy(pts_hbm.at[0, b, pl.ds(po, PER_TILE)], xv)
        pltpu.sync_copy(pts_hbm.at[1, b, pl.ds(po, PER_TILE)], yv)

        map_off = b_local * MAP

        def ibody(i, carry):
            x16 = xv[pl.ds(i * _L, _L)]
            y16 = yv[pl.ds(i * _L, _L)]
            xi = (x16 * W).astype(jnp.int32)
            yi = (y16 * H).astype(jnp.int32)
            ivm[i // (NIDX // _L), pl.ds((i % (NIDX // _L)) * _L, _L)] = (
                map_off + yi * W + xi
            )
            return carry

        lax.fori_loop(0, PER_TILE // _L, ibody, 0)

        def zbody(i, carry):
            zv[pl.ds(i * _L, _L)] = jnp.zeros((_L,), jnp.float32)
            return carry

        lax.fori_loop(0, ZCH // _L, zbody, 0)

        def obody(i, carry):
            ones_v[pl.ds(i * _L, _L)] = jnp.ones((_L,), jnp.float32)
            return carry

        lax.fori_loop(0, NIDX // _L, obody, 0)
        base = s * SLICE
        for k in range(SLICE // ZCH):
            pltpu.sync_copy(zv, smap.at[pl.ds(base + k * ZCH, ZCH)])
        plsc.subcore_barrier()

        for j in range(PER_TILE // NIDX):
            pltpu.sync_copy(ones_v, smap.at[ivm.at[j]], add=True)
        plsc.subcore_barrier()

        out_base = c * CORE_MAP + base
        pltpu.sync_copy(
            smap.at[pl.ds(base, SLICE)], out_hbm.at[pl.ds(out_base, SLICE)]
        )

    return sc_scatter(pts_t)


def _t_build_body(t_ref, *, H, half, inv_z):
    r = lax.broadcasted_iota(jnp.int32, (H, H), 0).astype(jnp.float32)
    cc = lax.broadcasted_iota(jnp.int32, (H, H), 1).astype(jnp.float32)
    d = r - cc
    inv_two_sigma2 = -1.0 / (2.0 * SIGMA * SIGMA)
    t_ref[...] = jnp.where(
        jnp.abs(d) <= half,
        jnp.exp(d * d * inv_two_sigma2) * inv_z,
        0.0,
    )


def _tc_build_t(H, half, inv_z):
    body = functools.partial(_t_build_body, H=H, half=half, inv_z=inv_z)
    return pl.pallas_call(
        body,
        out_shape=jax.ShapeDtypeStruct((H, H), jnp.float32),
    )()


def _tc_blur_body(t_ref, m_ref, o_ref):
    T = t_ref[...]
    A = lax.dot_general(
        T, m_ref[0], (((1,), (0,)), ((), ())),
        preferred_element_type=jnp.float32,
    )
    o_ref[0] = lax.dot_general(
        A, T, (((1,), (0,)), ((), ())),
        preferred_element_type=jnp.float32,
    )


def _tc_blur(t_mat, counts, B, H, W):
    return pl.pallas_call(
        _tc_blur_body,
        grid=(B,),
        in_specs=[
            pl.BlockSpec((H, H), lambda b: (0, 0)),
            pl.BlockSpec((1, H, W), lambda b: (b, 0, 0)),
        ],
        out_specs=pl.BlockSpec((1, H, W), lambda b: (b, 0, 0)),
        out_shape=jax.ShapeDtypeStruct((B, H, W), jnp.float32),
    )(t_mat, counts)


def kernel(feature_map, points):
    B, C, H, W = feature_map.shape
    P = points.shape[1]
    ks, half, inv_z = _kernel_consts(min(H, W))

    pts_t = jnp.transpose(points, (2, 0, 1))

    t_mat = _tc_build_t(H, half, inv_z)
    counts = _sc_scatter_counts(pts_t, B, P, H, W).reshape(B, H, W)
    return _tc_blur(t_mat, counts, B, H, W)


# speedup vs baseline: 1.3282x; 1.0333x over previous
import functools
import math

import jax
import jax.numpy as jnp
from jax import lax
from jax.experimental import pallas as pl
from jax.experimental.pallas import tpu as pltpu
from jax.experimental.pallas import tpu_sc as plsc

KERNEL_SIZE_FACTOR = 0.1
SIGMA = 3.0


def _kernel_consts(H):
    ks = int(H * KERNEL_SIZE_FACTOR)
    if ks % 2 == 0:
        ks += 1
    half = ks // 2
    c = (ks - 1) / 2.0
    z = sum(math.exp(-((i - c) ** 2) / (2.0 * SIGMA**2)) for i in range(ks))
    return ks, half, 1.0 / z


_NC = 2
_NS = 16
_L = 16


def _sc_scatter_counts(pts_t, B, P, H, W):
    MAP = H * W
    BPC = B // _NC
    TPB = _NS // BPC
    ROWS = H // TPB
    SLICE = ROWS * W

    mesh = plsc.VectorSubcoreMesh(
        core_axis_name="c", subcore_axis_name="s"
    )

    @functools.partial(
        pl.kernel,
        out_type=jax.ShapeDtypeStruct((B * MAP,), jnp.float32),
        mesh=mesh,
        scratch_types=[
            pltpu.VMEM((P,), jnp.float32),
            pltpu.VMEM((P,), jnp.float32),
            pltpu.VMEM((SLICE,), jnp.float32),
        ],
        compiler_params=pltpu.CompilerParams(needs_layout_passes=False),
    )
    def sc_scatter(pts_hbm, out_hbm, xv, yv, cslice):
        c = lax.axis_index("c")
        s = lax.axis_index("s")
        b = c * BPC + s // TPB
        r0 = (s % TPB) * ROWS

        pltpu.sync_copy(pts_hbm.at[0, b], xv)
        pltpu.sync_copy(pts_hbm.at[1, b], yv)

        zeros = jnp.zeros((_L,), jnp.float32)

        def zbody(i, carry):
            o = i * (4 * _L)
            cslice[pl.ds(o, _L)] = zeros
            cslice[pl.ds(o + _L, _L)] = zeros
            cslice[pl.ds(o + 2 * _L, _L)] = zeros
            cslice[pl.ds(o + 3 * _L, _L)] = zeros
            return carry

        lax.fori_loop(0, SLICE // (4 * _L), zbody, 0)

        ones = jnp.ones((_L,), jnp.float32)

        def sbody(i, carry):
            x16 = xv[pl.ds(i * _L, _L)]
            y16 = yv[pl.ds(i * _L, _L)]
            xi = (x16 * W).astype(jnp.int32)
            yi = (y16 * H).astype(jnp.int32) - r0
            mask = (yi >= 0) & (yi < ROWS)
            idx = jnp.where(mask, yi * W + xi, 0)
            plsc.addupdate_scatter(cslice, [idx], ones, mask=mask)
            return carry

        lax.fori_loop(0, P // _L, sbody, 0)

        pltpu.sync_copy(
            cslice, out_hbm.at[pl.ds(b * MAP + r0 * W, SLICE)]
        )

    return sc_scatter(pts_t)


def _t_build_body(t_ref, *, H, half, inv_z):
    r = lax.broadcasted_iota(jnp.int32, (H, H), 0).astype(jnp.float32)
    cc = lax.broadcasted_iota(jnp.int32, (H, H), 1).astype(jnp.float32)
    d = r - cc
    inv_two_sigma2 = -1.0 / (2.0 * SIGMA * SIGMA)
    t_ref[...] = jnp.where(
        jnp.abs(d) <= half,
        jnp.exp(d * d * inv_two_sigma2) * inv_z,
        0.0,
    )


def _tc_build_t(H, half, inv_z):
    body = functools.partial(_t_build_body, H=H, half=half, inv_z=inv_z)
    return pl.pallas_call(
        body,
        out_shape=jax.ShapeDtypeStruct((H, H), jnp.float32),
    )()


def _tc_blur_body(t_ref, m_ref, o_ref):
    T = t_ref[...]
    A = lax.dot_general(
        T, m_ref[0], (((1,), (0,)), ((), ())),
        preferred_element_type=jnp.float32,
    )
    o_ref[0] = lax.dot_general(
        A, T, (((1,), (0,)), ((), ())),
        preferred_element_type=jnp.float32,
    )


def _tc_blur(t_mat, counts, B, H, W):
    return pl.pallas_call(
        _tc_blur_body,
        grid=(B,),
        in_specs=[
            pl.BlockSpec((H, H), lambda b: (0, 0)),
            pl.BlockSpec((1, H, W), lambda b: (b, 0, 0)),
        ],
        out_specs=pl.BlockSpec((1, H, W), lambda b: (b, 0, 0)),
        out_shape=jax.ShapeDtypeStruct((B, H, W), jnp.float32),
    )(t_mat, counts)


def kernel(feature_map, points):
    B, C, H, W = feature_map.shape
    P = points.shape[1]
    ks, half, inv_z = _kernel_consts(min(H, W))

    pts_t = jnp.transpose(points, (2, 0, 1))

    t_mat = _tc_build_t(H, half, inv_z)
    counts = _sc_scatter_counts(pts_t, B, P, H, W).reshape(B, H, W)
    return _tc_blur(t_mat, counts, B, H, W)


# speedup vs baseline: 1.5748x; 1.1857x over previous
import functools
import math

import jax
import jax.numpy as jnp
from jax import lax
from jax.experimental import pallas as pl
from jax.experimental.pallas import tpu as pltpu
from jax.experimental.pallas import tpu_sc as plsc

KERNEL_SIZE_FACTOR = 0.1
SIGMA = 3.0


def _kernel_consts(H):
    ks = int(H * KERNEL_SIZE_FACTOR)
    if ks % 2 == 0:
        ks += 1
    half = ks // 2
    c = (ks - 1) / 2.0
    z = sum(math.exp(-((i - c) ** 2) / (2.0 * SIGMA**2)) for i in range(ks))
    return ks, half, 1.0 / z


_NC = 2
_NS = 16
_L = 16


def _sc_scatter_counts(pts_t, B, P, H, W):
    MAP = H * W
    BPC = B // _NC
    TPB = _NS // BPC
    ROWS = H // TPB
    SLICE = ROWS * W

    mesh = plsc.VectorSubcoreMesh(
        core_axis_name="c", subcore_axis_name="s"
    )

    @functools.partial(
        pl.kernel,
        out_type=jax.ShapeDtypeStruct((B * H, W), jnp.float32),
        mesh=mesh,
        scratch_types=[
            pltpu.VMEM((P,), jnp.float32),
            pltpu.VMEM((P,), jnp.float32),
            pltpu.VMEM((ROWS, W), jnp.float32),
        ],
        compiler_params=pltpu.CompilerParams(needs_layout_passes=False),
    )
    def sc_scatter(pts_hbm, out_hbm, xv, yv, cslice):
        c = lax.axis_index("c")
        s = lax.axis_index("s")
        b = c * BPC + s // TPB
        r0 = (s % TPB) * ROWS

        pltpu.sync_copy(pts_hbm.at[0, b], xv)
        pltpu.sync_copy(pts_hbm.at[1, b], yv)

        zeros = jnp.zeros((_L,), jnp.float32)

        def zbody(i, carry):
            for k in range(W // _L):
                cslice[i, pl.ds(k * _L, _L)] = zeros
            return carry

        lax.fori_loop(0, ROWS, zbody, 0)

        ones = jnp.ones((_L,), jnp.float32)

        def sbody(i, carry):
            x16 = xv[pl.ds(i * _L, _L)]
            y16 = yv[pl.ds(i * _L, _L)]
            xi = (x16 * W).astype(jnp.int32)
            yi = (y16 * H).astype(jnp.int32) - r0
            mask = (yi >= 0) & (yi < ROWS)
            yis = jnp.where(mask, yi, 0)
            plsc.addupdate_scatter(cslice, [yis, xi], ones, mask=mask)
            return carry

        lax.fori_loop(0, P // _L, sbody, 0)

        pltpu.sync_copy(cslice, out_hbm.at[pl.ds(b * H + r0, ROWS), :])

    return sc_scatter(pts_t)


def _t_build_body(t_ref, *, H, half, inv_z):
    r = lax.broadcasted_iota(jnp.int32, (H, H), 0).astype(jnp.float32)
    cc = lax.broadcasted_iota(jnp.int32, (H, H), 1).astype(jnp.float32)
    d = r - cc
    inv_two_sigma2 = -1.0 / (2.0 * SIGMA * SIGMA)
    t_ref[...] = jnp.where(
        jnp.abs(d) <= half,
        jnp.exp(d * d * inv_two_sigma2) * inv_z,
        0.0,
    )


def _tc_build_t(H, half, inv_z):
    body = functools.partial(_t_build_body, H=H, half=half, inv_z=inv_z)
    return pl.pallas_call(
        body,
        out_shape=jax.ShapeDtypeStruct((H, H), jnp.float32),
    )()


def _tc_blur_body(t_ref, cnt_hbm, o_ref, m_vmem, a_vmem, *, B, H, W):
    pltpu.sync_copy(cnt_hbm, m_vmem)
    T = t_ref[...]
    a_vmem[...] = lax.dot_general(
        m_vmem[...], T, (((1,), (0,)), ((), ())),
        preferred_element_type=jnp.float32,
    )
    for b in range(B):
        o_ref[b] = lax.dot_general(
            T, a_vmem[pl.ds(b * H, H), :], (((1,), (0,)), ((), ())),
            preferred_element_type=jnp.float32,
        )


def _tc_blur(t_mat, counts_flat, B, H, W):
    body = functools.partial(_tc_blur_body, B=B, H=H, W=W)
    return pl.pallas_call(
        body,
        in_specs=[
            pl.BlockSpec((H, H), lambda: (0, 0)),
            pl.BlockSpec(memory_space=pl.ANY),
        ],
        out_specs=pl.BlockSpec((B, H, W), lambda: (0, 0, 0)),
        out_shape=jax.ShapeDtypeStruct((B, H, W), jnp.float32),
        scratch_shapes=[
            pltpu.VMEM((B * H, W), jnp.float32),
            pltpu.VMEM((B * H, W), jnp.float32),
        ],
    )(t_mat, counts_flat)


def kernel(feature_map, points):
    B, C, H, W = feature_map.shape
    P = points.shape[1]
    ks, half, inv_z = _kernel_consts(min(H, W))

    pts_t = jnp.transpose(points, (2, 0, 1))

    t_mat = _tc_build_t(H, half, inv_z)
    counts_flat = _sc_scatter_counts(pts_t, B, P, H, W)
    return _tc_blur(t_mat, counts_flat, B, H, W)
